# Initial kernel scaffold; baseline (speedup 1.0000x reference)
import jax
import jax.numpy as jnp
from jax.experimental import pallas as pl


def kernel(X_transformed, time_context, edge_score_now, edge_score_lag, prior_adj, W1, b1, W2, b2):
    raise NotImplementedError("write your pallas kernel here")


# speedup vs baseline: 1.3786x; 1.3786x over previous
import functools

import jax
import jax.numpy as jnp
from jax.experimental import pallas as pl
from jax.experimental.pallas import tpu as pltpu


def _body(T, H, B, N, x_ref, now_ref, lag_ref, prior_ref, p_ref, b2_ref,
          now_out, lag_out):
    t = pl.program_id(0)
    x = x_ref[0]
    feats = jnp.sum(x, axis=0) * (1.0 / H)
    mu = jnp.sum(feats, axis=0, keepdims=True) * (1.0 / B)
    c = feats - mu
    num = jax.lax.dot_general(c, c, (((0,), (0,)), ((), ())),
                              preferred_element_type=jnp.float32)
    sq = jnp.sum(c * c, axis=0)
    den = jax.lax.rsqrt(sq[:, None] * sq[None, :] + 1e-8)
    corr = jnp.minimum(jnp.abs(num) * den, 1.0)

    is0 = (t == 0).astype(jnp.float32)
    param = now_ref[...] * is0 + lag_ref[...] * (1.0 - is0)

    acc = jnp.zeros_like(corr)
    for k in range(16):
        h = corr * p_ref[0, k] + param * p_ref[1, k] + p_ref[2, k]
        h = jnp.maximum(h, 0.01 * h)
        acc = acc + p_ref[3, k] * h
    s = jax.nn.sigmoid(acc + b2_ref[0])

    rows = jax.lax.broadcasted_iota(jnp.int32, (N, N), 0)
    cols = jax.lax.broadcasted_iota(jnp.int32, (N, N), 1)
    mask = (rows != cols).astype(jnp.float32)

    w_t = 1.0 - (0.9 / (T - 1)) * t.astype(jnp.float32)
    mean_w_lag = sum(1.0 - 0.9 * i / (T - 1) for i in range(1, T)) / (T - 1)

    @pl.when(t == 0)
    def _():
        now_out[...] = mask * (0.7 * s + 0.3 * jax.nn.sigmoid(prior_ref[...]))

    @pl.when(t == 1)
    def _():
        lag_out[...] = w_t * s

    @pl.when(jnp.logical_and(t >= 2, t <= T - 2))
    def _():
        lag_out[...] = lag_out[...] + w_t * s

    @pl.when(t == T - 1)
    def _():
        tot = lag_out[...] + w_t * s
        lag_out[...] = mask * ((0.7 / (T - 1)) * tot
                               + (0.3 * mean_w_lag)
                               * jax.nn.sigmoid(prior_ref[...]))


def kernel(X_transformed, time_context, edge_score_now, edge_score_lag,
           prior_adj, W1, b1, W2, b2):
    T, H, B, N = X_transformed.shape
    params = jnp.stack([W1[:, 0], W1[:, 1], b1, W2[0, :]], axis=0)
    b2s = jnp.reshape(b2, (1,)).astype(jnp.float32)

    body = functools.partial(_body, T, H, B, N)
    out = pl.pallas_call(
        body,
        grid=(T,),
        in_specs=[
            pl.BlockSpec((1, H, B, N), lambda t: (t, 0, 0, 0)),
            pl.BlockSpec((N, N), lambda t: (0, 0)),
            pl.BlockSpec((N, N), lambda t: (0, 0)),
            pl.BlockSpec((N, N), lambda t: (0, 0)),
            pl.BlockSpec(memory_space=pltpu.SMEM),
            pl.BlockSpec(memory_space=pltpu.SMEM),
        ],
        out_specs=[
            pl.BlockSpec((N, N), lambda t: (0, 0)),
            pl.BlockSpec((N, N), lambda t: (0, 0)),
        ],
        out_shape=[
            jax.ShapeDtypeStruct((N, N), jnp.float32),
            jax.ShapeDtypeStruct((N, N), jnp.float32),
        ],
        compiler_params=pltpu.CompilerParams(
            dimension_semantics=("arbitrary",)),
    )(X_transformed, edge_score_now, edge_score_lag, prior_adj, params, b2s)
    return (out[0], out[1])
